# Initial kernel scaffold; baseline (speedup 1.0000x reference)
import jax
import jax.numpy as jnp
from jax.experimental import pallas as pl


def kernel(x_1, edge_index_1, x_2, edge_index_2, W1a, b1a, W2a, b2a, W1b, b1b, W2b, b2b, Wm, bm, Wf, bf):
    raise NotImplementedError("write your pallas kernel here")


# speedup vs baseline: 15.0972x; 15.0972x over previous
import jax
import jax.numpy as jnp
from jax import lax
from jax.experimental import pallas as pl
from jax.experimental.pallas import tpu as pltpu
from jax.experimental.pallas import tpu_sc as plsc

_N = 10000
_E = 320000
_D = 128

_TILES = 16
_EPT = _E // _TILES
_CHUNK = 128
_NFULL = _EPT // _CHUNK
_TAIL = _EPT - _NFULL * _CHUNK

_mesh = plsc.VectorSubcoreMesh(core_axis_name="c", subcore_axis_name="s")


def _fill_ones(ref, n):
    for j in range(n // 16):
        ref[pl.ds(16 * j, 16)] = jnp.ones((16,), jnp.float32)


def _deg_body(dst1, dst2, zeros_n, deg1_out, deg2_out,
              deg_sh, idx_v, ones_v, idx_t, ones_t):
    c = lax.axis_index("c")
    s = lax.axis_index("s")
    _fill_ones(ones_v, _CHUNK)
    _fill_ones(ones_t, _TAIL)

    @pl.when(s == 0)
    def _():
        pltpu.sync_copy(zeros_n, deg_sh)

    plsc.subcore_barrier()
    base = s * _EPT

    def count(dst_hbm):
        def body(i, carry):
            off = pl.multiple_of(base + i * _CHUNK, 8)
            pltpu.sync_copy(dst_hbm.at[pl.ds(off, _CHUNK)], idx_v)
            pltpu.sync_copy(ones_v, deg_sh.at[idx_v], add=True)
            return carry

        lax.fori_loop(0, _NFULL, body, 0)
        offt = pl.multiple_of(base + _NFULL * _CHUNK, 8)
        pltpu.sync_copy(dst_hbm.at[pl.ds(offt, _TAIL)], idx_t)
        pltpu.sync_copy(ones_t, deg_sh.at[idx_t], add=True)

    @pl.when(c == 0)
    def _():
        count(dst1)

    @pl.when(c == 1)
    def _():
        count(dst2)

    plsc.subcore_barrier()

    @pl.when((s == 0) & (c == 0))
    def _():
        pltpu.sync_copy(deg_sh, deg1_out)

    @pl.when((s == 0) & (c == 1))
    def _():
        pltpu.sync_copy(deg_sh, deg2_out)


def _agg_body(g1, g2, src1, dst1, src2, dst2, zeros_nd, acc1_out, acc2_out,
              acc_sh, isrc, idst, rows, isrc_t, idst_t, rows_t):
    c = lax.axis_index("c")
    s = lax.axis_index("s")

    @pl.when(s == 0)
    def _():
        pltpu.sync_copy(zeros_nd, acc_sh)

    plsc.subcore_barrier()
    base = s * _EPT

    def aggregate(g_hbm, src_hbm, dst_hbm):
        def body(i, carry):
            off = pl.multiple_of(base + i * _CHUNK, 8)
            pltpu.sync_copy(src_hbm.at[pl.ds(off, _CHUNK)], isrc)
            pltpu.sync_copy(dst_hbm.at[pl.ds(off, _CHUNK)], idst)
            pltpu.sync_copy(g_hbm.at[isrc], rows)
            pltpu.sync_copy(rows, acc_sh.at[idst], add=True)
            return carry

        lax.fori_loop(0, _NFULL, body, 0)
        offt = pl.multiple_of(base + _NFULL * _CHUNK, 8)
        pltpu.sync_copy(src_hbm.at[pl.ds(offt, _TAIL)], isrc_t)
        pltpu.sync_copy(dst_hbm.at[pl.ds(offt, _TAIL)], idst_t)
        pltpu.sync_copy(g_hbm.at[isrc_t], rows_t)
        pltpu.sync_copy(rows_t, acc_sh.at[idst_t], add=True)

    @pl.when(c == 0)
    def _():
        aggregate(g1, src1, dst1)

    @pl.when(c == 1)
    def _():
        aggregate(g2, src2, dst2)

    plsc.subcore_barrier()

    @pl.when((s == 0) & (c == 0))
    def _():
        pltpu.sync_copy(acc_sh, acc1_out)

    @pl.when((s == 0) & (c == 1))
    def _():
        pltpu.sync_copy(acc_sh, acc2_out)


_deg_call = pl.kernel(
    _deg_body,
    out_type=(jax.ShapeDtypeStruct((_N,), jnp.float32),
              jax.ShapeDtypeStruct((_N,), jnp.float32)),
    mesh=_mesh,
    scratch_types=[
        pltpu.VMEM_SHARED((_N,), jnp.float32),
        pltpu.VMEM((_CHUNK,), jnp.int32),
        pltpu.VMEM((_CHUNK,), jnp.float32),
        pltpu.VMEM((_TAIL,), jnp.int32),
        pltpu.VMEM((_TAIL,), jnp.float32),
    ],
)

_agg_call = pl.kernel(
    _agg_body,
    out_type=(jax.ShapeDtypeStruct((_N, _D), jnp.float32),
              jax.ShapeDtypeStruct((_N, _D), jnp.float32)),
    mesh=_mesh,
    scratch_types=[
        pltpu.VMEM_SHARED((_N, _D), jnp.float32),
        pltpu.VMEM((_CHUNK,), jnp.int32),
        pltpu.VMEM((_CHUNK,), jnp.int32),
        pltpu.VMEM((_CHUNK, _D), jnp.float32),
        pltpu.VMEM((_TAIL,), jnp.int32),
        pltpu.VMEM((_TAIL,), jnp.int32),
        pltpu.VMEM((_TAIL, _D), jnp.float32),
    ],
)


def _elu(v):
    return jnp.where(v > 0, v, jnp.exp(v) - 1.0)


def _pre_body(x1, i1, w1, x2, i2, w2, g1o, g2o):
    d1 = lax.rsqrt(i1[...] + 2.0)
    g1o[...] = d1 * jnp.dot(x1[...], w1[...], preferred_element_type=jnp.float32)
    d2 = lax.rsqrt(i2[...] + 2.0)
    g2o[...] = d2 * jnp.dot(x2[...], w2[...], preferred_element_type=jnp.float32)


def _mid_body(a1, g1, i1, b1, w1, a2, g2, i2, b2, w2, g1o, g2o):
    d1 = lax.rsqrt(i1[...] + 2.0)
    h1 = _elu(d1 * a1[...] + 2.0 * d1 * g1[...] + b1[...])
    g1o[...] = d1 * jnp.dot(h1, w1[...], preferred_element_type=jnp.float32)
    d2 = lax.rsqrt(i2[...] + 2.0)
    h2 = _elu(d2 * a2[...] + 2.0 * d2 * g2[...] + b2[...])
    g2o[...] = d2 * jnp.dot(h2, w2[...], preferred_element_type=jnp.float32)


def _fin_body(a1, g1, i1, b1, a2, g2, i2, b2, wm, bm, wf, bf, out):
    d1 = lax.rsqrt(i1[...] + 2.0)
    h1 = _elu(d1 * a1[...] + 2.0 * d1 * g1[...] + b1[...])
    d2 = lax.rsqrt(i2[...] + 2.0)
    h2 = _elu(d2 * a2[...] + 2.0 * d2 * g2[...] + b2[...])
    m = (jnp.dot(h1, wm[0:_D, :], preferred_element_type=jnp.float32)
         + jnp.dot(h2, wm[_D:2 * _D, :], preferred_element_type=jnp.float32)
         + bm[...])
    out[...] = jax.nn.sigmoid(
        jnp.dot(m, wf[...], preferred_element_type=jnp.float32) + bf[...])


def _tc_call(body, n_out, out_dims):
    return pl.pallas_call(
        body,
        out_shape=tuple(jax.ShapeDtypeStruct((_N, d), jnp.float32)
                        for d in out_dims[:n_out]),
    )


_pre = _tc_call(_pre_body, 2, (_D, _D))
_mid = _tc_call(_mid_body, 2, (_D, _D))
_fin = _tc_call(_fin_body, 1, (16,))


def kernel(x_1, edge_index_1, x_2, edge_index_2,
           W1a, b1a, W2a, b2a, W1b, b1b, W2b, b2b, Wm, bm, Wf, bf):
    src1, dst1 = edge_index_1[0], edge_index_1[1]
    src2, dst2 = edge_index_2[0], edge_index_2[1]
    zeros_n = jnp.zeros((_N,), jnp.float32)
    zeros_nd = jnp.zeros((_N, _D), jnp.float32)

    ideg1, ideg2 = _deg_call(dst1, dst2, zeros_n)
    i1 = ideg1.reshape(_N, 1)
    i2 = ideg2.reshape(_N, 1)

    ga, gc = _pre(x_1, i1, W1a, x_2, i2, W1b)
    accA, accC = _agg_call(ga, gc, src1, dst1, src2, dst2, zeros_nd)
    gb, gd = _mid(accA, ga, i1, b1a.reshape(1, -1), W2a,
                  accC, gc, i2, b1b.reshape(1, -1), W2b)
    accB, accD = _agg_call(gb, gd, src1, dst1, src2, dst2, zeros_nd)
    (out,) = _fin(accB, gb, i1, b2a.reshape(1, -1),
                  accD, gd, i2, b2b.reshape(1, -1),
                  Wm, bm.reshape(1, -1), Wf, bf.reshape(1, -1))
    return out


# speedup vs baseline: 23.3099x; 1.5440x over previous
import jax
import jax.numpy as jnp
from jax import lax
from jax.experimental import pallas as pl
from jax.experimental.pallas import tpu as pltpu
from jax.experimental.pallas import tpu_sc as plsc

_N = 10000
_E = 320000
_D = 128

_TILES = 16
_EPT = _E // _TILES
_CHUNK = 128
_NFULL = _EPT // _CHUNK
_TAIL = _EPT - _NFULL * _CHUNK
_AC = 80
_AN = _EPT // _AC
_AG = _AN // 2

_mesh = plsc.VectorSubcoreMesh(core_axis_name="c", subcore_axis_name="s")


def _fill_ones(ref, n):
    for j in range(n // 16):
        ref[pl.ds(16 * j, 16)] = jnp.ones((16,), jnp.float32)


def _deg_body(d1m, d1t, d2m, d2t, zeros_n, deg1_out, deg2_out,
              deg_sh, idxm, idxt, ones_v, ones_t, sem_deg):
    c = lax.axis_index("c")
    s = lax.axis_index("s")
    _fill_ones(ones_v, _CHUNK)
    _fill_ones(ones_t, _TAIL)

    @pl.when(s == 0)
    def _():
        pltpu.sync_copy(zeros_n, deg_sh)

    @pl.when(c == 0)
    def _():
        pltpu.sync_copy(d1m.at[s], idxm)
        pltpu.sync_copy(d1t.at[s], idxt)

    @pl.when(c == 1)
    def _():
        pltpu.sync_copy(d2m.at[s], idxm)
        pltpu.sync_copy(d2t.at[s], idxt)

    plsc.subcore_barrier()

    def fire(ch, carry):
        pltpu.async_copy(ones_v, deg_sh.at[idxm.at[ch]], sem_deg, add=True)
        return carry

    def drain(ch, carry):
        pltpu.make_async_copy(ones_v, deg_sh.at[idxm.at[0]], sem_deg).wait()
        return carry

    lax.fori_loop(0, _NFULL, fire, 0)
    pltpu.sync_copy(ones_t, deg_sh.at[idxt], add=True)
    lax.fori_loop(0, _NFULL, drain, 0)

    plsc.subcore_barrier()

    @pl.when((s == 0) & (c == 0))
    def _():
        pltpu.sync_copy(deg_sh, deg1_out)

    @pl.when((s == 0) & (c == 1))
    def _():
        pltpu.sync_copy(deg_sh, deg2_out)


def _agg_body(g1, g2, p1, p2, zeros_nd, acc1_out, acc2_out,
              acc_sh, packed, isrc0, idst0, isrc1, idst1, rows0, rows1,
              gsem0, gsem1, ssem0, ssem1):
    c = lax.axis_index("c")
    s = lax.axis_index("s")

    @pl.when(s == 0)
    def _():
        pltpu.sync_copy(zeros_nd, acc_sh)

    @pl.when(c == 0)
    def _():
        pltpu.sync_copy(p1.at[s], packed)

    @pl.when(c == 1)
    def _():
        pltpu.sync_copy(p2.at[s], packed)

    plsc.subcore_barrier()

    def unpack(ch, isrc, idst):
        for j in range(_AC // 16):
            p = packed[pl.ds(ch * _AC + 16 * j, 16)]
            isrc[pl.ds(16 * j, 16)] = p & 0xFFFF
            idst[pl.ds(16 * j, 16)] = lax.shift_right_logical(p, 16)

    def aggregate(g_hbm):
        def gstart(isrc, rows, sem):
            pltpu.async_copy(g_hbm.at[isrc], rows, sem)

        def gwait(rows, sem):
            pltpu.make_async_copy(g_hbm.at[isrc0], rows, sem).wait()

        def sstart(idst, rows, sem):
            pltpu.async_copy(rows, acc_sh.at[idst], sem, add=True)

        def swait(rows, sem):
            pltpu.make_async_copy(rows, acc_sh.at[idst0], sem).wait()

        unpack(0, isrc0, idst0)
        gstart(isrc0, rows0, gsem0)

        def body(g, carry):
            i0 = 2 * g
            gwait(rows0, gsem0)

            @pl.when(g > 0)
            def _():
                swait(rows1, ssem1)

            unpack(i0 + 1, isrc1, idst1)
            gstart(isrc1, rows1, gsem1)
            sstart(idst0, rows0, ssem0)
            gwait(rows1, gsem1)
            swait(rows0, ssem0)

            @pl.when(g < _AG - 1)
            def _():
                unpack(i0 + 2, isrc0, idst0)
                gstart(isrc0, rows0, gsem0)

            sstart(idst1, rows1, ssem1)
            return carry

        lax.fori_loop(0, _AG, body, 0)
        swait(rows1, ssem1)

    @pl.when(c == 0)
    def _():
        aggregate(g1)

    @pl.when(c == 1)
    def _():
        aggregate(g2)

    plsc.subcore_barrier()

    @pl.when((s == 0) & (c == 0))
    def _():
        pltpu.sync_copy(acc_sh, acc1_out)

    @pl.when((s == 0) & (c == 1))
    def _():
        pltpu.sync_copy(acc_sh, acc2_out)


_deg_call = pl.kernel(
    _deg_body,
    out_type=(jax.ShapeDtypeStruct((_N,), jnp.float32),
              jax.ShapeDtypeStruct((_N,), jnp.float32)),
    mesh=_mesh,
    scratch_types=[
        pltpu.VMEM_SHARED((_N,), jnp.float32),
        pltpu.VMEM((_NFULL, _CHUNK), jnp.int32),
        pltpu.VMEM((_TAIL,), jnp.int32),
        pltpu.VMEM((_CHUNK,), jnp.float32),
        pltpu.VMEM((_TAIL,), jnp.float32),
        pltpu.SemaphoreType.DMA,
    ],
)

_agg_call = pl.kernel(
    _agg_body,
    out_type=(jax.ShapeDtypeStruct((_N, _D), jnp.float32),
              jax.ShapeDtypeStruct((_N, _D), jnp.float32)),
    mesh=_mesh,
    scratch_types=[
        pltpu.VMEM_SHARED((_N, _D), jnp.float32),
        pltpu.VMEM((_EPT,), jnp.int32),
        pltpu.VMEM((_AC,), jnp.int32),
        pltpu.VMEM((_AC,), jnp.int32),
        pltpu.VMEM((_AC,), jnp.int32),
        pltpu.VMEM((_AC,), jnp.int32),
        pltpu.VMEM((_AC, _D), jnp.float32),
        pltpu.VMEM((_AC, _D), jnp.float32),
        pltpu.SemaphoreType.DMA,
        pltpu.SemaphoreType.DMA,
        pltpu.SemaphoreType.DMA,
        pltpu.SemaphoreType.DMA,
    ],
)


def _elu(v):
    return jnp.where(v > 0, v, jnp.exp(v) - 1.0)


def _pre_body(x1, i1, w1, x2, i2, w2, g1o, g2o):
    d1 = lax.rsqrt(i1[...] + 2.0)
    g1o[...] = d1 * jnp.dot(x1[...], w1[...], preferred_element_type=jnp.float32)
    d2 = lax.rsqrt(i2[...] + 2.0)
    g2o[...] = d2 * jnp.dot(x2[...], w2[...], preferred_element_type=jnp.float32)


def _mid_body(a1, g1, i1, b1, w1, a2, g2, i2, b2, w2, g1o, g2o):
    d1 = lax.rsqrt(i1[...] + 2.0)
    h1 = _elu(d1 * a1[...] + 2.0 * d1 * g1[...] + b1[...])
    g1o[...] = d1 * jnp.dot(h1, w1[...], preferred_element_type=jnp.float32)
    d2 = lax.rsqrt(i2[...] + 2.0)
    h2 = _elu(d2 * a2[...] + 2.0 * d2 * g2[...] + b2[...])
    g2o[...] = d2 * jnp.dot(h2, w2[...], preferred_element_type=jnp.float32)


def _fin_body(a1, g1, i1, b1, a2, g2, i2, b2, wm, bm, wf, bf, out):
    d1 = lax.rsqrt(i1[...] + 2.0)
    h1 = _elu(d1 * a1[...] + 2.0 * d1 * g1[...] + b1[...])
    d2 = lax.rsqrt(i2[...] + 2.0)
    h2 = _elu(d2 * a2[...] + 2.0 * d2 * g2[...] + b2[...])
    m = (jnp.dot(h1, wm[0:_D, :], preferred_element_type=jnp.float32)
         + jnp.dot(h2, wm[_D:2 * _D, :], preferred_element_type=jnp.float32)
         + bm[...])
    out[...] = jax.nn.sigmoid(
        jnp.dot(m, wf[...], preferred_element_type=jnp.float32) + bf[...])


def _tc_call(body, n_out, out_dims):
    return pl.pallas_call(
        body,
        out_shape=tuple(jax.ShapeDtypeStruct((_N, d), jnp.float32)
                        for d in out_dims[:n_out]),
    )


_pre = _tc_call(_pre_body, 2, (_D, _D))
_mid = _tc_call(_mid_body, 2, (_D, _D))
_fin = _tc_call(_fin_body, 1, (16,))


def _split_idx(v):
    m = v.reshape(_TILES, _EPT)
    main = m[:, :_NFULL * _CHUNK].reshape(_TILES, _NFULL, _CHUNK)
    tail = m[:, _NFULL * _CHUNK:]
    return main, tail


def kernel(x_1, edge_index_1, x_2, edge_index_2,
           W1a, b1a, W2a, b2a, W1b, b1b, W2b, b2b, Wm, bm, Wf, bf):
    d1m, d1t = _split_idx(edge_index_1[1])
    d2m, d2t = _split_idx(edge_index_2[1])
    p1 = (edge_index_1[0] | (edge_index_1[1] << 16)).reshape(_TILES, _EPT)
    p2 = (edge_index_2[0] | (edge_index_2[1] << 16)).reshape(_TILES, _EPT)
    zeros_n = jnp.zeros((_N,), jnp.float32)
    zeros_nd = jnp.zeros((_N, _D), jnp.float32)

    ideg1, ideg2 = _deg_call(d1m, d1t, d2m, d2t, zeros_n)
    i1 = ideg1.reshape(_N, 1)
    i2 = ideg2.reshape(_N, 1)

    ga, gc = _pre(x_1, i1, W1a, x_2, i2, W1b)
    accA, accC = _agg_call(ga, gc, p1, p2, zeros_nd)
    gb, gd = _mid(accA, ga, i1, b1a.reshape(1, -1), W2a,
                  accC, gc, i2, b1b.reshape(1, -1), W2b)
    accB, accD = _agg_call(gb, gd, p1, p2, zeros_nd)
    (out,) = _fin(accB, gb, i1, b2a.reshape(1, -1),
                  accD, gd, i2, b2b.reshape(1, -1),
                  Wm, bm.reshape(1, -1), Wf, bf.reshape(1, -1))
    return out
